# Initial kernel scaffold; baseline (speedup 1.0000x reference)
import jax
import jax.numpy as jnp
from jax.experimental import pallas as pl


def kernel(x, lstm_w_ih, lstm_w_hh, lstm_b_ih, lstm_b_hh, halt_w, halt_b, gate_w, gate_b, w1, b1, w2, b2, ln_g, ln_b):
    raise NotImplementedError("write your pallas kernel here")


# speedup vs baseline: 1.0725x; 1.0725x over previous
import jax
import jax.numpy as jnp
from jax import lax
from jax.experimental import pallas as pl
from jax.experimental.pallas import tpu as pltpu

H = 1024
D = 1024
OUT = 1024
E = 8
EP = 128
STEPS = 4
THRESH = 1.0 - 0.01


def _dotT(a, b, precision=lax.Precision.HIGHEST):
    return lax.dot_general(a, b, (((1,), (1,)), ((), ())),
                           preferred_element_type=jnp.float32,
                           precision=precision)


def _dotT_bf16(a, b):
    return lax.dot_general(a.astype(jnp.bfloat16), b.astype(jnp.bfloat16),
                           (((1,), (1,)), ((), ())),
                           preferred_element_type=jnp.float32)


def _lstm_kernel(x_ref, wih_ref, whh_ref, bih_ref, bhh_ref, hw_ref, hb_ref,
                 hstack_ref, hp_ref, colmin_ref):
    ti = pl.program_id(0)
    x = x_ref[...]
    u = _dotT_bf16(x, wih_ref[...]) + bih_ref[...]
    bt = x.shape[0]
    h = jnp.zeros((bt, H), jnp.float32)
    c = jnp.zeros((bt, H), jnp.float32)
    hp = jnp.zeros((bt, 1), jnp.float32)
    hp_all = jnp.zeros((bt, EP), jnp.float32)
    lane = lax.broadcasted_iota(jnp.int32, (1, EP), 1)
    hw_bf = hw_ref[...].astype(jnp.bfloat16).astype(jnp.float32)
    for s in range(STEPS):
        gates = (u + _dotT_bf16(h, whh_ref[...])) + bhh_ref[...]
        ig = jax.nn.sigmoid(gates[:, 0:H])
        fg = jax.nn.sigmoid(gates[:, H:2 * H])
        gg = jnp.tanh(gates[:, 2 * H:3 * H])
        og = jax.nn.sigmoid(gates[:, 3 * H:4 * H])
        c = fg * c + ig * gg
        h = og * jnp.tanh(c)
        hbf = h.astype(jnp.bfloat16).astype(jnp.float32)
        y = jax.nn.sigmoid(
            jnp.sum(hbf * hw_bf, axis=1, keepdims=True) + hb_ref[...])
        hp = hp + y * (1.0 - hp)
        hstack_ref[s] = h
        hp_all = jnp.where(lane == s, hp, hp_all)
    hp_ref[...] = hp_all
    bmin = jnp.min(hp_all, axis=0, keepdims=True)

    @pl.when(ti == 0)
    def _():
        colmin_ref[...] = bmin

    @pl.when(ti != 0)
    def _():
        colmin_ref[...] = jnp.minimum(colmin_ref[...], bmin)


def _gate(t, gw_ref, gb_ref, w_ref):
    lane = lax.broadcasted_iota(jnp.int32, (1, EP), 1)
    logits = _dotT_bf16(t, gw_ref[...]) + gb_ref[...]
    m1 = jnp.max(logits, axis=1, keepdims=True)
    i1 = jnp.min(jnp.where(logits == m1, lane, EP), axis=1, keepdims=True)
    mask1 = lane == i1
    logits2 = jnp.where(mask1, -1e30, logits)
    m2 = jnp.max(logits2, axis=1, keepdims=True)
    i2 = jnp.min(jnp.where(logits2 == m2, lane, EP), axis=1, keepdims=True)
    mask2 = lane == i2
    ew = jnp.exp(m2 - m1)
    denom = 1.0 + ew
    w_ref[...] = (jnp.where(mask1, 1.0 / denom, 0.0)
                  + jnp.where(mask2, ew / denom, 0.0))


def _gate_kernel(t_ref, gw_ref, gb_ref, w_ref):
    _gate(t_ref[...], gw_ref, gb_ref, w_ref)


def _combine_kernel(hstack_ref, hp_ref, colmin_ref, gw_ref, gb_ref,
                    t_ref, w_ref):
    lane = lax.broadcasted_iota(jnp.int32, (1, EP), 1)
    cvec = jnp.where(colmin_ref[...] <= THRESH, 1.0, 0.0)
    notc = 1.0 - cvec
    jj = lax.broadcasted_iota(jnp.int32, (EP, EP), 0)
    ss = lax.broadcasted_iota(jnp.int32, (EP, EP), 1)
    ltri = jnp.where(jj < ss, 1.0, 0.0)
    pref = lax.dot_general(notc, ltri, (((1,), (0,)), ((), ())),
                           preferred_element_type=jnp.float32,
                           precision=lax.Precision.HIGHEST)
    a = jnp.where((pref == 0.0) & (lane < STEPS), 1.0, 0.0)
    n_upd = jnp.sum(a)
    hp = hp_ref[...]
    rem = jnp.sum(a * (1.0 - hp), axis=1, keepdims=True)
    hsum = jnp.zeros_like(hstack_ref[0])
    for s in range(STEPS):
        a_s = jnp.sum(jnp.where(lane == s, a, 0.0))
        hsum = hsum + a_s * (hstack_ref[s] * rem)
    t = hsum / n_upd
    t_ref[...] = t
    _gate(t, gw_ref, gb_ref, w_ref)


def _dotT_bf16(a, b):
    return lax.dot_general(a.astype(jnp.bfloat16), b.astype(jnp.bfloat16),
                           (((1,), (1,)), ((), ())),
                           preferred_element_type=jnp.float32)


def _expert_contrib(t_ref, w_ref, w1_ref, b1_ref, w2_ref, b2_ref, e):
    lane = lax.broadcasted_iota(jnp.int32, (1, EP), 1)
    we = jnp.sum(jnp.where(lane == e, w_ref[...], 0.0), axis=1, keepdims=True)
    hid = jnp.maximum(_dotT_bf16(t_ref[...], w1_ref[0]) + b1_ref[0], 0.0)
    o = _dotT_bf16(hid, w2_ref[0]) + b2_ref[0]
    return we * o


def _moe_acc_kernel(t_ref, w_ref, w1_ref, b1_ref, w2_ref, b2_ref,
                    out_ref, acc_ref):
    e = pl.program_id(1)
    contrib = _expert_contrib(t_ref, w_ref, w1_ref, b1_ref, w2_ref, b2_ref, e)

    @pl.when(e == 0)
    def _():
        acc_ref[...] = contrib

    @pl.when(e != 0)
    def _():
        acc_ref[...] += contrib

    @pl.when(e == E - 1)
    def _():
        out_ref[...] = acc_ref[...]


def _moe_acc_ln_kernel(t_ref, w_ref, w1_ref, b1_ref, w2_ref, b2_ref,
                       g_ref, bb_ref, out_ref, acc_ref):
    e = pl.program_id(1)
    contrib = _expert_contrib(t_ref, w_ref, w1_ref, b1_ref, w2_ref, b2_ref, e)

    @pl.when(e == 0)
    def _():
        acc_ref[...] = contrib

    @pl.when(e != 0)
    def _():
        acc_ref[...] += contrib

    @pl.when(e == E - 1)
    def _():
        tt = acc_ref[...]
        mu = jnp.mean(tt, axis=1, keepdims=True)
        var = jnp.mean((tt - mu) ** 2, axis=1, keepdims=True)
        out_ref[...] = (tt - mu) / jnp.sqrt(var + 1e-5) * g_ref[...] + bb_ref[...]


def kernel(x, lstm_w_ih, lstm_w_hh, lstm_b_ih, lstm_b_hh, halt_w, halt_b,
           gate_w, gate_b, w1, b1, w2, b2, ln_g, ln_b):
    B = x.shape[0]
    f32 = jnp.float32

    bih = lstm_b_ih.reshape(1, 4 * H)
    bhh = lstm_b_hh.reshape(1, 4 * H)
    hb = halt_b.reshape(1, 1)
    gwp = jnp.zeros((2, EP, D), f32).at[:, :E].set(gate_w)
    gbp = jnp.full((2, 1, EP), -1e30, f32).at[:, 0, :E].set(gate_b)

    BTL = 128
    ntl = B // BTL
    hstack, hp_all, colmin = pl.pallas_call(
        _lstm_kernel,
        grid=(ntl,),
        in_specs=[
            pl.BlockSpec((BTL, D), lambda i: (i, 0)),
            pl.BlockSpec((4 * H, D), lambda i: (0, 0)),
            pl.BlockSpec((4 * H, H), lambda i: (0, 0)),
            pl.BlockSpec((1, 4 * H), lambda i: (0, 0)),
            pl.BlockSpec((1, 4 * H), lambda i: (0, 0)),
            pl.BlockSpec((1, H), lambda i: (0, 0)),
            pl.BlockSpec((1, 1), lambda i: (0, 0)),
        ],
        out_specs=[
            pl.BlockSpec((STEPS, BTL, H), lambda i: (0, i, 0)),
            pl.BlockSpec((BTL, EP), lambda i: (i, 0)),
            pl.BlockSpec((1, EP), lambda i: (0, 0)),
        ],
        out_shape=[
            jax.ShapeDtypeStruct((STEPS, B, H), f32),
            jax.ShapeDtypeStruct((B, EP), f32),
            jax.ShapeDtypeStruct((1, EP), f32),
        ],
        compiler_params=pltpu.CompilerParams(
            dimension_semantics=("arbitrary",)),
    )(x, lstm_w_ih, lstm_w_hh, bih, bhh, halt_w, hb)

    BT = 256
    nt = B // BT
    t1, W1 = pl.pallas_call(
        _combine_kernel,
        grid=(nt,),
        in_specs=[
            pl.BlockSpec((STEPS, BT, H), lambda i: (0, i, 0)),
            pl.BlockSpec((BT, EP), lambda i: (i, 0)),
            pl.BlockSpec((1, EP), lambda i: (0, 0)),
            pl.BlockSpec((EP, D), lambda i: (0, 0)),
            pl.BlockSpec((1, EP), lambda i: (0, 0)),
        ],
        out_specs=[
            pl.BlockSpec((BT, D), lambda i: (i, 0)),
            pl.BlockSpec((BT, EP), lambda i: (i, 0)),
        ],
        out_shape=[
            jax.ShapeDtypeStruct((B, D), f32),
            jax.ShapeDtypeStruct((B, EP), f32),
        ],
        compiler_params=pltpu.CompilerParams(
            dimension_semantics=("arbitrary",)),
    )(hstack, hp_all, colmin, gwp[0], gbp[0])

    moe_in_specs = [
        pl.BlockSpec((BT, D), lambda i, e: (i, 0)),
        pl.BlockSpec((BT, EP), lambda i, e: (i, 0)),
        pl.BlockSpec((1, H, D), lambda i, e: (e, 0, 0)),
        pl.BlockSpec((1, 1, H), lambda i, e: (e, 0, 0)),
        pl.BlockSpec((1, OUT, H), lambda i, e: (e, 0, 0)),
        pl.BlockSpec((1, 1, OUT), lambda i, e: (e, 0, 0)),
    ]
    t2 = pl.pallas_call(
        _moe_acc_kernel,
        grid=(nt, E),
        in_specs=moe_in_specs,
        out_specs=pl.BlockSpec((BT, OUT), lambda i, e: (i, 0)),
        out_shape=jax.ShapeDtypeStruct((B, OUT), f32),
        scratch_shapes=[pltpu.VMEM((BT, OUT), f32)],
        compiler_params=pltpu.CompilerParams(
            dimension_semantics=("arbitrary", "arbitrary")),
    )(t1, W1, w1[0], b1[0][:, None, :], w2[0], b2[0][:, None, :])

    W2 = pl.pallas_call(
        _gate_kernel,
        grid=(nt,),
        in_specs=[
            pl.BlockSpec((BT, D), lambda i: (i, 0)),
            pl.BlockSpec((EP, D), lambda i: (0, 0)),
            pl.BlockSpec((1, EP), lambda i: (0, 0)),
        ],
        out_specs=pl.BlockSpec((BT, EP), lambda i: (i, 0)),
        out_shape=jax.ShapeDtypeStruct((B, EP), f32),
    )(t2, gwp[1], gbp[1])

    out = pl.pallas_call(
        _moe_acc_ln_kernel,
        grid=(nt, E),
        in_specs=moe_in_specs + [
            pl.BlockSpec((1, OUT), lambda i, e: (0, 0)),
            pl.BlockSpec((1, OUT), lambda i, e: (0, 0)),
        ],
        out_specs=pl.BlockSpec((BT, OUT), lambda i, e: (i, 0)),
        out_shape=jax.ShapeDtypeStruct((B, OUT), f32),
        scratch_shapes=[pltpu.VMEM((BT, OUT), f32)],
        compiler_params=pltpu.CompilerParams(
            dimension_semantics=("arbitrary", "arbitrary")),
    )(t2, W2, w1[1], b1[1][:, None, :], w2[1], b2[1][:, None, :],
      ln_g.reshape(1, OUT), ln_b.reshape(1, OUT))
    return out


# speedup vs baseline: 1.3960x; 1.3016x over previous
import jax
import jax.numpy as jnp
from jax import lax
from jax.experimental import pallas as pl
from jax.experimental.pallas import tpu as pltpu

H = 1024
D = 1024
OUT = 1024
E = 8
EP = 128
STEPS = 4
THRESH = 1.0 - 0.01


def _dotT(a, b, precision=lax.Precision.HIGHEST):
    return lax.dot_general(a, b, (((1,), (1,)), ((), ())),
                           preferred_element_type=jnp.float32,
                           precision=precision)


def _dotT_bf16(a, b):
    return lax.dot_general(a.astype(jnp.bfloat16), b.astype(jnp.bfloat16),
                           (((1,), (1,)), ((), ())),
                           preferred_element_type=jnp.float32)


def _lstm_kernel(x_ref, wih_ref, whh_ref, bih_ref, bhh_ref, hw_ref, hb_ref,
                 hstack_ref, hp_ref, colmin_ref):
    ti = pl.program_id(0)
    x = x_ref[...]
    u = _dotT_bf16(x, wih_ref[...]) + bih_ref[...]
    bt = x.shape[0]
    h = jnp.zeros((bt, H), jnp.float32)
    c = jnp.zeros((bt, H), jnp.float32)
    hp = jnp.zeros((bt, 1), jnp.float32)
    hp_all = jnp.zeros((bt, EP), jnp.float32)
    lane = lax.broadcasted_iota(jnp.int32, (1, EP), 1)
    hw_bf = hw_ref[...].astype(jnp.bfloat16).astype(jnp.float32)
    for s in range(STEPS):
        gates = (u + _dotT_bf16(h, whh_ref[...])) + bhh_ref[...]
        ig = jax.nn.sigmoid(gates[:, 0:H])
        fg = jax.nn.sigmoid(gates[:, H:2 * H])
        gg = jnp.tanh(gates[:, 2 * H:3 * H])
        og = jax.nn.sigmoid(gates[:, 3 * H:4 * H])
        c = fg * c + ig * gg
        h = og * jnp.tanh(c)
        hbf = h.astype(jnp.bfloat16).astype(jnp.float32)
        y = jax.nn.sigmoid(
            jnp.sum(hbf * hw_bf, axis=1, keepdims=True) + hb_ref[...])
        hp = hp + y * (1.0 - hp)
        hstack_ref[s] = h
        hp_all = jnp.where(lane == s, hp, hp_all)
    hp_ref[...] = hp_all
    bmin = jnp.min(hp_all, axis=0, keepdims=True)

    @pl.when(ti == 0)
    def _():
        colmin_ref[...] = bmin

    @pl.when(ti != 0)
    def _():
        colmin_ref[...] = jnp.minimum(colmin_ref[...], bmin)


def _gate(t, gw_ref, gb_ref, w_ref):
    lane = lax.broadcasted_iota(jnp.int32, (1, EP), 1)
    logits = _dotT_bf16(t, gw_ref[...]) + gb_ref[...]
    m1 = jnp.max(logits, axis=1, keepdims=True)
    i1 = jnp.min(jnp.where(logits == m1, lane, EP), axis=1, keepdims=True)
    mask1 = lane == i1
    logits2 = jnp.where(mask1, -1e30, logits)
    m2 = jnp.max(logits2, axis=1, keepdims=True)
    i2 = jnp.min(jnp.where(logits2 == m2, lane, EP), axis=1, keepdims=True)
    mask2 = lane == i2
    ew = jnp.exp(m2 - m1)
    denom = 1.0 + ew
    w_ref[...] = (jnp.where(mask1, 1.0 / denom, 0.0)
                  + jnp.where(mask2, ew / denom, 0.0))


def _gate_kernel(t_ref, gw_ref, gb_ref, w_ref):
    _gate(t_ref[...], gw_ref, gb_ref, w_ref)


def _combine_kernel(hstack_ref, hp_ref, colmin_ref, gw_ref, gb_ref,
                    t_ref, w_ref):
    lane = lax.broadcasted_iota(jnp.int32, (1, EP), 1)
    cvec = jnp.where(colmin_ref[...] <= THRESH, 1.0, 0.0)
    notc = 1.0 - cvec
    jj = lax.broadcasted_iota(jnp.int32, (EP, EP), 0)
    ss = lax.broadcasted_iota(jnp.int32, (EP, EP), 1)
    ltri = jnp.where(jj < ss, 1.0, 0.0)
    pref = lax.dot_general(notc, ltri, (((1,), (0,)), ((), ())),
                           preferred_element_type=jnp.float32,
                           precision=lax.Precision.HIGHEST)
    a = jnp.where((pref == 0.0) & (lane < STEPS), 1.0, 0.0)
    n_upd = jnp.sum(a)
    hp = hp_ref[...]
    rem = jnp.sum(a * (1.0 - hp), axis=1, keepdims=True)
    hsum = jnp.zeros_like(hstack_ref[0])
    for s in range(STEPS):
        a_s = jnp.sum(jnp.where(lane == s, a, 0.0))
        hsum = hsum + a_s * (hstack_ref[s] * rem)
    t = hsum / n_upd
    t_ref[...] = t
    _gate(t, gw_ref, gb_ref, w_ref)


def _expert_contrib(t_ref, w_ref, w1_ref, b1_ref, w2_ref, b2_ref, e):
    lane = lax.broadcasted_iota(jnp.int32, (1, EP), 1)
    we = jnp.sum(jnp.where(lane == e, w_ref[...], 0.0), axis=1, keepdims=True)
    hid = jnp.maximum(_dotT_bf16(t_ref[...], w1_ref[0]) + b1_ref[0], 0.0)
    o = _dotT_bf16(hid, w2_ref[0]) + b2_ref[0]
    return we * o


def _moe_acc_kernel(t_ref, w_ref, w1_ref, b1_ref, w2_ref, b2_ref,
                    out_ref, acc_ref):
    e = pl.program_id(1)
    contrib = _expert_contrib(t_ref, w_ref, w1_ref, b1_ref, w2_ref, b2_ref, e)

    @pl.when(e == 0)
    def _():
        acc_ref[...] = contrib

    @pl.when(e != 0)
    def _():
        acc_ref[...] += contrib

    @pl.when(e == E - 1)
    def _():
        out_ref[...] = acc_ref[...]


def _moe_acc_ln_kernel(t_ref, w_ref, w1_ref, b1_ref, w2_ref, b2_ref,
                       g_ref, bb_ref, out_ref, acc_ref):
    e = pl.program_id(1)
    contrib = _expert_contrib(t_ref, w_ref, w1_ref, b1_ref, w2_ref, b2_ref, e)

    @pl.when(e == 0)
    def _():
        acc_ref[...] = contrib

    @pl.when(e != 0)
    def _():
        acc_ref[...] += contrib

    @pl.when(e == E - 1)
    def _():
        tt = acc_ref[...]
        mu = jnp.mean(tt, axis=1, keepdims=True)
        var = jnp.mean((tt - mu) ** 2, axis=1, keepdims=True)
        out_ref[...] = (tt - mu) / jnp.sqrt(var + 1e-5) * g_ref[...] + bb_ref[...]


def kernel(x, lstm_w_ih, lstm_w_hh, lstm_b_ih, lstm_b_hh, halt_w, halt_b,
           gate_w, gate_b, w1, b1, w2, b2, ln_g, ln_b):
    B = x.shape[0]
    f32 = jnp.float32

    bih = lstm_b_ih.reshape(1, 4 * H)
    bhh = lstm_b_hh.reshape(1, 4 * H)
    hb = halt_b.reshape(1, 1)
    gwp = jnp.zeros((2, EP, D), f32).at[:, :E].set(gate_w)
    gbp = jnp.full((2, 1, EP), -1e30, f32).at[:, 0, :E].set(gate_b)

    BTL = 128
    ntl = B // BTL
    hstack, hp_all, colmin = pl.pallas_call(
        _lstm_kernel,
        grid=(ntl,),
        in_specs=[
            pl.BlockSpec((BTL, D), lambda i: (i, 0)),
            pl.BlockSpec((4 * H, D), lambda i: (0, 0)),
            pl.BlockSpec((4 * H, H), lambda i: (0, 0)),
            pl.BlockSpec((1, 4 * H), lambda i: (0, 0)),
            pl.BlockSpec((1, 4 * H), lambda i: (0, 0)),
            pl.BlockSpec((1, H), lambda i: (0, 0)),
            pl.BlockSpec((1, 1), lambda i: (0, 0)),
        ],
        out_specs=[
            pl.BlockSpec((STEPS, BTL, H), lambda i: (0, i, 0)),
            pl.BlockSpec((BTL, EP), lambda i: (i, 0)),
            pl.BlockSpec((1, EP), lambda i: (0, 0)),
        ],
        out_shape=[
            jax.ShapeDtypeStruct((STEPS, B, H), f32),
            jax.ShapeDtypeStruct((B, EP), f32),
            jax.ShapeDtypeStruct((1, EP), f32),
        ],
        compiler_params=pltpu.CompilerParams(
            dimension_semantics=("arbitrary",)),
    )(x, lstm_w_ih, lstm_w_hh, bih, bhh, halt_w, hb)

    BT = 256
    nt = B // BT
    t1, W1 = pl.pallas_call(
        _combine_kernel,
        grid=(nt,),
        in_specs=[
            pl.BlockSpec((STEPS, BT, H), lambda i: (0, i, 0)),
            pl.BlockSpec((BT, EP), lambda i: (i, 0)),
            pl.BlockSpec((1, EP), lambda i: (0, 0)),
            pl.BlockSpec((EP, D), lambda i: (0, 0)),
            pl.BlockSpec((1, EP), lambda i: (0, 0)),
        ],
        out_specs=[
            pl.BlockSpec((BT, D), lambda i: (i, 0)),
            pl.BlockSpec((BT, EP), lambda i: (i, 0)),
        ],
        out_shape=[
            jax.ShapeDtypeStruct((B, D), f32),
            jax.ShapeDtypeStruct((B, EP), f32),
        ],
        compiler_params=pltpu.CompilerParams(
            dimension_semantics=("arbitrary",)),
    )(hstack, hp_all, colmin, gwp[0], gbp[0])

    BTM = 1024
    ntm = B // BTM
    moe_in_specs = [
        pl.BlockSpec((BTM, D), lambda i, e: (i, 0)),
        pl.BlockSpec((BTM, EP), lambda i, e: (i, 0)),
        pl.BlockSpec((1, H, D), lambda i, e: (e, 0, 0)),
        pl.BlockSpec((1, 1, H), lambda i, e: (e, 0, 0)),
        pl.BlockSpec((1, OUT, H), lambda i, e: (e, 0, 0)),
        pl.BlockSpec((1, 1, OUT), lambda i, e: (e, 0, 0)),
    ]
    t2 = pl.pallas_call(
        _moe_acc_kernel,
        grid=(ntm, E),
        in_specs=moe_in_specs,
        out_specs=pl.BlockSpec((BTM, OUT), lambda i, e: (i, 0)),
        out_shape=jax.ShapeDtypeStruct((B, OUT), f32),
        scratch_shapes=[pltpu.VMEM((BTM, OUT), f32)],
        compiler_params=pltpu.CompilerParams(
            dimension_semantics=("arbitrary", "arbitrary")),
    )(t1, W1, w1[0], b1[0][:, None, :], w2[0], b2[0][:, None, :])

    W2 = pl.pallas_call(
        _gate_kernel,
        grid=(nt,),
        in_specs=[
            pl.BlockSpec((BT, D), lambda i: (i, 0)),
            pl.BlockSpec((EP, D), lambda i: (0, 0)),
            pl.BlockSpec((1, EP), lambda i: (0, 0)),
        ],
        out_specs=pl.BlockSpec((BT, EP), lambda i: (i, 0)),
        out_shape=jax.ShapeDtypeStruct((B, EP), f32),
    )(t2, gwp[1], gbp[1])

    out = pl.pallas_call(
        _moe_acc_ln_kernel,
        grid=(ntm, E),
        in_specs=moe_in_specs + [
            pl.BlockSpec((1, OUT), lambda i, e: (0, 0)),
            pl.BlockSpec((1, OUT), lambda i, e: (0, 0)),
        ],
        out_specs=pl.BlockSpec((BTM, OUT), lambda i, e: (i, 0)),
        out_shape=jax.ShapeDtypeStruct((B, OUT), f32),
        scratch_shapes=[pltpu.VMEM((BTM, OUT), f32)],
        compiler_params=pltpu.CompilerParams(
            dimension_semantics=("arbitrary", "arbitrary")),
    )(t2, W2, w1[1], b1[1][:, None, :], w2[1], b2[1][:, None, :],
      ln_g.reshape(1, OUT), ln_b.reshape(1, OUT))
    return out
